# baseline (speedup 1.0000x reference)
import functools

import numpy as np

import jax
import jax.numpy as jnp
from jax import lax
from jax.experimental import pallas as pl
from jax.experimental.pallas import tpu as pltpu
from jax.experimental.pallas import tpu_sc as plsc

_N = 10000
_F = 128
_E = 320000
_TILES = 16
_CORES = 2
_NP = 10112
_ROWS_PT = _NP // _TILES
_CH = 128
_CHUNKS = 80
_HB = 40
_SLOTS = 4
_EPAD = _CH * _CHUNKS * _TILES * _CORES
_PAD_NP = (_N + np.arange(_EPAD - _E) % (_NP - _N)).astype(np.int32)


def _zero_acc_slice(table, acc, s):
    base = s * _ROWS_PT
    for i in range(6):
        pltpu.sync_copy(table.at[pl.ds(_N, 96)],
                        acc.at[pl.ds(base + i * 96, 96)])
    pltpu.sync_copy(table.at[pl.ds(_N, _ROWS_PT - 576)],
                    acc.at[pl.ds(base + 576, _ROWS_PT - 576)])


def _seg_sum_body(table, src1, dst1, bnd_s, bnd_d, out, sidx, didx64,
                  arena, acc, gs0, gs1, gs2, gs3, ss0, ss1, ss2, ss3):
    c = lax.axis_index("c")
    s = lax.axis_index("s")

    _zero_acc_slice(table, acc, s)
    plsc.subcore_barrier()

    slots = [arena.at[pl.ds(k * 64, 64)] for k in range(_SLOTS)]
    gs = [gs0, gs1, gs2, gs3]
    ss = [ss0, ss1, ss2, ss3]
    subs = 2 * _HB

    def gather_src(t):
        return table.at[sidx.at[pl.ds(t * 64, 64)]]

    def fire_gather(t, k):
        pltpu.async_copy(gather_src(t), slots[k], gs[k])

    def wait_gather(t, k):
        pltpu.make_async_copy(gather_src(t), slots[k], gs[k]).wait()

    def fire_scatter(t, k):
        pltpu.async_copy(slots[k], acc.at[didx64.at[t]], ss[k], add=True)

    def wait_scatter(t, k):
        pltpu.make_async_copy(slots[k], acc.at[didx64.at[t]], ss[k]).wait()

    bbase = (_E // _CH) // _HB * _HB

    def load_ids(arr, bnd, r0b):
        @pl.when(r0b < bbase)
        def _():
            pltpu.sync_copy(arr.at[pl.ds(r0b * _CH, _HB * _CH)], sidx)

        @pl.when(r0b == bbase)
        def _():
            pltpu.sync_copy(bnd.at[pl.ds(0, _HB * _CH)], sidx)

        @pl.when(r0b == bbase + _HB)
        def _():
            pltpu.sync_copy(bnd.at[pl.ds(_HB * _CH, _HB * _CH)], sidx)

    row0 = (c * _TILES + s) * _CHUNKS
    for blk in range(_CHUNKS // _HB):
        load_ids(dst1, bnd_d, row0 + blk * _HB)

        def conv(r2, _):
            for g in range(4):
                didx64[r2, pl.ds(g * 16, 16)] = sidx[pl.ds(r2 * 64 + g * 16, 16)]
            return 0

        lax.fori_loop(0, subs, conv, 0)
        load_ids(src1, bnd_s, row0 + blk * _HB)

        for k in range(_SLOTS):
            fire_gather(k, k)

        def batch(i, _):
            t0 = _SLOTS * i
            for k in range(_SLOTS):
                wait_gather(t0 + k, k)
                fire_scatter(t0 + k, k)
            for k in range(_SLOTS):
                t = t0 + k
                wait_scatter(t, k)

                @pl.when(t + _SLOTS < subs)
                def _():
                    fire_gather(t + _SLOTS, k)

            return 0

        lax.fori_loop(0, subs // _SLOTS, batch, 0)

    plsc.subcore_barrier()
    pltpu.sync_copy(acc.at[pl.ds(s * _ROWS_PT, _ROWS_PT)],
                    out.at[pl.ds(c * _NP + s * _ROWS_PT, _ROWS_PT)])


@functools.cache
def _get_seg_sum():
    mesh = plsc.VectorSubcoreMesh(core_axis_name="c", subcore_axis_name="s")
    return pl.kernel(
        _seg_sum_body,
        out_type=(jax.ShapeDtypeStruct((_CORES * _NP, _F), jnp.float32),),
        mesh=mesh,
        scratch_types=(
            pltpu.VMEM((_HB * _CH,), jnp.int32),
            pltpu.VMEM((2 * _HB, 64), jnp.int32),
            pltpu.VMEM((_SLOTS * 64, _F), jnp.float32),
            pltpu.VMEM_SHARED((_NP, _F), jnp.float32),
            pltpu.SemaphoreType.DMA,
            pltpu.SemaphoreType.DMA,
            pltpu.SemaphoreType.DMA,
            pltpu.SemaphoreType.DMA,
            pltpu.SemaphoreType.DMA,
            pltpu.SemaphoreType.DMA,
            pltpu.SemaphoreType.DMA,
            pltpu.SemaphoreType.DMA,
        ),
    )


def _psum(p):
    return p[0:_N] + p[_NP:_NP + _N]


def _prep_body(x_ref, xh_ref, xsq_ref):
    x = x_ref[...]
    norm = jnp.sqrt(jnp.sum(x * x, axis=1, keepdims=True))
    xh = x / jnp.maximum(norm, 1e-8)
    pad = jnp.zeros((_NP - _N, _F), jnp.float32)
    xhp = jnp.concatenate([xh, pad], axis=0)
    xh_ref[...] = xhp
    xsq_ref[...] = xhp * xhp


def _prep(x):
    sds = jax.ShapeDtypeStruct((_NP, _F), jnp.float32)
    return pl.pallas_call(_prep_body, out_shape=(sds, sds))(x)


def _colstats(v):
    m = jnp.mean(v, axis=0, keepdims=True)
    var = jnp.sum((v - m) * (v - m), axis=0, keepdims=True) / (v.shape[0] - 1)
    s = jnp.maximum(jnp.sqrt(var), 1e-8)
    return m, s


def _gate_pre_body(x_ref, gW1_ref, gb1_ref, gW2_ref, gb2_ref, gates_ref):
    x = x_ref[...]
    xm, xs = _colstats(x)
    xn = (x - xm) / xs
    g1 = jnp.maximum(
        jnp.dot(xn, gW1_ref[...], preferred_element_type=jnp.float32)
        + gb1_ref[...], 0.0)
    gates_ref[...] = jax.nn.sigmoid(
        jnp.dot(g1, gW2_ref[...], preferred_element_type=jnp.float32)
        + gb2_ref[...])


def _gate_pre(x, gW1, gb1, gW2, gb2):
    return pl.pallas_call(
        _gate_pre_body,
        out_shape=jax.ShapeDtypeStruct((_N, _F), jnp.float32),
    )(x, gW1, gb1, gW2, gb2)


def _gate_post_body(xh_ref, p1_ref, p2_ref, gates_ref, faW1_ref,
                    fab1_ref, faW2_ref, fab2_ref, h0_ref, degc_ref):
    xh = xh_ref[pl.ds(0, _N), :]
    s1 = _psum(p1_ref[...])
    s2 = _psum(p2_ref[...])
    deg = jnp.sum(s2, axis=1, keepdims=True)
    degc_ref[...] = jnp.maximum(deg, 1.0)
    agg = deg * xh * xh - 2.0 * xh * s1 + s2
    r_normal = agg / (deg + 1e-12)
    r_flip = 2.0 - r_normal
    gates = gates_ref[...]

    rm, rs = _colstats(r_normal)
    rn = (r_normal - rm) / rs
    rf = (r_flip - rm) / rs
    z = gates * rn + (1.0 - gates) * rf
    zm, zs = _colstats(z)
    en = (z - zm) / zs
    a1 = jnp.maximum(
        jnp.dot(en, faW1_ref[...], preferred_element_type=jnp.float32)
        + fab1_ref[...], 0.0)
    attn = jax.nn.sigmoid(
        jnp.dot(a1, faW2_ref[...], preferred_element_type=jnp.float32)
        + fab2_ref[...])
    h0 = en * attn
    pad = jnp.zeros((_NP - _N, _F), jnp.float32)
    h0_ref[...] = jnp.concatenate([h0, pad], axis=0)


def _gate_post(xhp, p1, p2, gates, faW1, fab1, faW2, fab2):
    return pl.pallas_call(
        _gate_post_body,
        out_shape=(jax.ShapeDtypeStruct((_NP, _F), jnp.float32),
                   jax.ShapeDtypeStruct((_N, 1), jnp.float32)),
    )(xhp, p1, p2, gates, faW1, fab1, faW2, fab2)


def _matmul_body(h_ref, W_ref, out_ref):
    out_ref[...] = jnp.dot(h_ref[...], W_ref[...],
                           preferred_element_type=jnp.float32)


def _matmul(h, W):
    return pl.pallas_call(
        _matmul_body,
        out_shape=jax.ShapeDtypeStruct((_NP, W.shape[1]), jnp.float32),
    )(h, W)


def _sage_post_body(hs_ref, pn_ref, degc_ref, Wn_ref, b_ref, out_ref):
    nsum = _psum(pn_ref[...])
    neigh = nsum / degc_ref[...]
    out = jnp.maximum(
        hs_ref[pl.ds(0, _N), :]
        + jnp.dot(neigh, Wn_ref[...], preferred_element_type=jnp.float32)
        + b_ref[...], 0.0)
    pad = jnp.zeros((_NP - _N, _F), jnp.float32)
    out_ref[...] = jnp.concatenate([out, pad], axis=0)


def _sage_post(hs, pn, degc, Wn, b):
    return pl.pallas_call(
        _sage_post_body,
        out_shape=jax.ShapeDtypeStruct((_NP, _F), jnp.float32),
    )(hs, pn, degc, Wn, b)


def _final_post_body(hs_ref, pn_ref, degc_ref, W3n_ref, cb3_ref, Wc_ref,
                     bc_ref, out_ref):
    nsum = _psum(pn_ref[...])
    neigh = nsum / degc_ref[...]
    h3 = jnp.maximum(
        hs_ref[pl.ds(0, _N), :]
        + jnp.dot(neigh, W3n_ref[...], preferred_element_type=jnp.float32)
        + cb3_ref[...], 0.0)
    out_ref[...] = (jnp.dot(h3, Wc_ref[...], preferred_element_type=jnp.float32)
                    + bc_ref[...])


def _final_post(hs, pn, degc, W3n, cb3, Wc, bc):
    return pl.pallas_call(
        _final_post_body,
        out_shape=jax.ShapeDtypeStruct((_N, 40), jnp.float32),
    )(hs, pn, degc, W3n, cb3, Wc, bc)


def kernel(features, edge_index, gW1, gb1, gW2, gb2, faW1, fab1, faW2, fab2,
           W1s, W1n, cb1, W2s, W2n, cb2, W3s, W3n, cb3, Wc, bc):
    src = edge_index[0]
    dst = edge_index[1]
    padv = jnp.asarray(_PAD_NP)
    bbase = (_E // _CH) // _HB * _HB
    bnd_s = jnp.concatenate([src[bbase * _CH:], padv])
    bnd_d = jnp.concatenate([dst[bbase * _CH:], padv])

    seg_sum = _get_seg_sum()

    xhp, xsqp = _prep(features)
    (p1,) = seg_sum(xhp, src, dst, bnd_s, bnd_d)
    (p2,) = seg_sum(xsqp, src, dst, bnd_s, bnd_d)
    gates = _gate_pre(features, gW1, gb1, gW2, gb2)
    h0, degc = _gate_post(xhp, p1, p2, gates, faW1, fab1, faW2, fab2)
    (p3,) = seg_sum(h0, src, dst, bnd_s, bnd_d)
    hs1 = _matmul(h0, W1s)
    h1 = _sage_post(hs1, p3, degc, W1n, cb1)
    (p4,) = seg_sum(h1, src, dst, bnd_s, bnd_d)
    hs2 = _matmul(h1, W2s)
    h2 = _sage_post(hs2, p4, degc, W2n, cb2)
    (p5,) = seg_sum(h2, src, dst, bnd_s, bnd_d)
    hs3 = _matmul(h2, W3s)
    return _final_post(hs3, p5, degc, W3n, cb3, Wc, bc)
